# baseline (speedup 1.0000x reference)
import functools

import jax
import jax.numpy as jnp
from jax import lax
from jax.experimental import pallas as pl
from jax.experimental.pallas import tpu as pltpu
from jax.experimental.pallas import tpu_sc as plsc

VOCAB = 1000000
MAXLEN = 200
EMBED = 64
BATCH = 4096

NUM_CORES = 2
NUM_SUBCORES = 16
NW = NUM_CORES * NUM_SUBCORES
TOTAL = BATCH * MAXLEN
PER_W = TOTAL // NW
CHUNK = MAXLEN
NCHUNK = PER_W // CHUNK
NBUF = 3
PREFETCH = 2
LANES = 16
VPR = EMBED // LANES
NGROUP = CHUNK // LANES


_mesh = plsc.VectorSubcoreMesh(core_axis_name="c", subcore_axis_name="s")


@functools.partial(
    pl.kernel,
    out_type=jax.ShapeDtypeStruct((TOTAL, EMBED), jnp.float32),
    mesh=_mesh,
    scratch_types=[
        pltpu.VMEM((PER_W,), jnp.int32),
        pltpu.VMEM((MAXLEN // 2, 2 * EMBED), jnp.float32),
        pltpu.VMEM((NBUF, CHUNK, EMBED), jnp.float32),
        pltpu.SemaphoreType.DMA((NBUF,)),
        pltpu.SemaphoreType.DMA((NBUF,)),
    ],
    compiler_params=pltpu.CompilerParams(use_tc_tiling_on_sc=True),
)
def _embed_sc(x_hbm, tok_hbm, pos_hbm, out_hbm, idx_v, pos_v, bufs, gsem, ssem):
    wid = lax.axis_index("s") * NUM_CORES + lax.axis_index("c")
    base = wid * PER_W

    pltpu.sync_copy(pos_hbm, pos_v)
    pltpu.sync_copy(x_hbm.at[pl.ds(base, PER_W)], idx_v)

    def fetch_group(g, b, u, row0, lo):
        tv = idx_v[pl.ds(g * CHUNK + row0, LANES)]
        for j in range(lo, LANES):
            tok = tv[j]
            pltpu.async_copy(tok_hbm.at[pl.ds(tok, 1), :],
                             bufs.at[b, pl.ds(row0 + j, 1), :],
                             gsem.at[b])

    def start_fetch(g, b):
        @pl.loop(0, NGROUP)
        def _(u):
            fetch_group(g, b, u, u * LANES, 0)

        fetch_group(g, b, NGROUP, CHUNK - LANES, LANES - (CHUNK % LANES))

    def wait_fetch(b):
        pltpu.make_async_copy(tok_hbm.at[pl.ds(0, CHUNK), :],
                              bufs.at[b], gsem.at[b]).wait()

    def start_store(g, b):
        pltpu.async_copy(
            bufs.at[b], out_hbm.at[pl.ds(base + g * CHUNK, CHUNK), :], ssem.at[b])

    def wait_store(g, b):
        pltpu.make_async_copy(
            bufs.at[b], out_hbm.at[pl.ds(base + g * CHUNK, CHUNK), :],
            ssem.at[b]).wait()

    def add_pos(b):
        @pl.loop(0, CHUNK // 2)
        def _(rr):
            for half in range(2):
                for v in range(VPR):
                    plsc.addupdate(
                        bufs.at[b, 2 * rr + half, pl.ds(v * LANES, LANES)],
                        pos_v[rr, pl.ds(half * EMBED + v * LANES, LANES)])

    for b in range(PREFETCH):
        start_fetch(b, b)

    NMAIN = (NCHUNK // NBUF) * NBUF

    @pl.loop(0, NCHUNK // NBUF)
    def _(i):
        for b in range(NBUF):
            g = i * NBUF + b
            h = g + PREFETCH
            bh = (b + PREFETCH) % NBUF

            @pl.when(h < NCHUNK)
            def _():
                @pl.when(h >= NBUF)
                def _():
                    wait_store(h - NBUF, bh)

                start_fetch(h, bh)

            wait_fetch(b)
            add_pos(b)
            start_store(g, b)

    for g in range(NMAIN, NCHUNK):
        b = g % NBUF
        wait_fetch(b)
        add_pos(b)
        start_store(g, b)

    for g in range(NCHUNK - NBUF, NCHUNK):
        wait_store(g, g % NBUF)


def kernel(x, token_table, pos_table):
    x_flat = x.reshape(TOTAL).astype(jnp.int32)
    pos2 = pos_table.reshape(MAXLEN // 2, 2 * EMBED)
    out = _embed_sc(x_flat, token_table, pos2)
    return out.reshape(BATCH, MAXLEN, EMBED)


# speedup vs baseline: 1.0895x; 1.0895x over previous
import functools

import jax
import jax.numpy as jnp
from jax import lax
from jax.experimental import pallas as pl
from jax.experimental.pallas import tpu as pltpu
from jax.experimental.pallas import tpu_sc as plsc

VOCAB = 1000000
MAXLEN = 200
EMBED = 64
BATCH = 4096

NUM_CORES = 2
NUM_SUBCORES = 16
NW = NUM_CORES * NUM_SUBCORES
TOTAL = BATCH * MAXLEN
PER_W = TOTAL // NW
CHUNK = 160
NCHUNK = PER_W // CHUNK
NBUF = 4
PREFETCH = 2
LANES = 16
VPR = EMBED // LANES
NGROUP = CHUNK // LANES


_mesh = plsc.VectorSubcoreMesh(core_axis_name="c", subcore_axis_name="s")


@functools.partial(
    pl.kernel,
    out_type=jax.ShapeDtypeStruct((TOTAL, EMBED), jnp.float32),
    mesh=_mesh,
    scratch_types=[
        pltpu.VMEM((PER_W,), jnp.int32),
        pltpu.VMEM((MAXLEN // 2, 2 * EMBED), jnp.float32),
        pltpu.VMEM((NBUF, CHUNK, EMBED), jnp.float32),
        pltpu.SemaphoreType.DMA((NBUF,)),
        pltpu.SemaphoreType.DMA((NBUF,)),
    ],
    compiler_params=pltpu.CompilerParams(use_tc_tiling_on_sc=True),
)
def _embed_sc(x_hbm, tok_hbm, pos_hbm, out_hbm, idx_v, pos_v, bufs, gsem, ssem):
    wid = lax.axis_index("s") * NUM_CORES + lax.axis_index("c")
    base = wid * PER_W

    pltpu.sync_copy(pos_hbm, pos_v)
    pltpu.sync_copy(x_hbm.at[pl.ds(base, PER_W)], idx_v)

    def fetch_group(g, b, u, row0, lo):
        tv = idx_v[pl.ds(g * CHUNK + row0, LANES)]
        for j in range(lo, LANES):
            tok = tv[j]
            pltpu.async_copy(tok_hbm.at[pl.ds(tok, 1), :],
                             bufs.at[b, pl.ds(row0 + j, 1), :],
                             gsem.at[b])

    def start_fetch(g, b):
        @pl.loop(0, NGROUP)
        def _(u):
            fetch_group(g, b, u, u * LANES, 0)

    def wait_fetch(b):
        pltpu.make_async_copy(tok_hbm.at[pl.ds(0, CHUNK), :],
                              bufs.at[b], gsem.at[b]).wait()

    def start_store(g, b):
        pltpu.async_copy(
            bufs.at[b], out_hbm.at[pl.ds(base + g * CHUNK, CHUNK), :], ssem.at[b])

    def wait_store(g, b):
        pltpu.make_async_copy(
            bufs.at[b], out_hbm.at[pl.ds(base + g * CHUNK, CHUNK), :],
            ssem.at[b]).wait()

    def add_pos(g, b):
        p0h = lax.rem(g * (CHUNK // 2), MAXLEN // 2)
        k1h = jnp.minimum((MAXLEN // 2) - p0h, CHUNK // 2)

        def body(rr, prow):
            for half in range(2):
                for v in range(VPR):
                    plsc.addupdate(
                        bufs.at[b, 2 * rr + half, pl.ds(v * LANES, LANES)],
                        pos_v[prow, pl.ds(half * EMBED + v * LANES, LANES)])

        @pl.loop(0, k1h)
        def _(rr):
            body(rr, p0h + rr)

        @pl.loop(k1h, CHUNK // 2)
        def _(rr):
            body(rr, p0h + rr - MAXLEN // 2)

    for b in range(PREFETCH):
        start_fetch(b, b)

    NMAIN = (NCHUNK // NBUF) * NBUF

    @pl.loop(0, NCHUNK // NBUF)
    def _(i):
        for b in range(NBUF):
            g = i * NBUF + b
            h = g + PREFETCH
            bh = (b + PREFETCH) % NBUF

            @pl.when(h < NCHUNK)
            def _():
                @pl.when(h >= NBUF)
                def _():
                    wait_store(h - NBUF, bh)

                start_fetch(h, bh)

            wait_fetch(b)
            add_pos(g, b)
            start_store(g, b)

    for g in range(NMAIN, NCHUNK):
        b = g % NBUF
        h = g + PREFETCH
        if h < NCHUNK:
            wait_store(h - NBUF, h % NBUF)
            start_fetch(h, h % NBUF)
        wait_fetch(b)
        add_pos(g, b)
        start_store(g, b)

    for g in range(NCHUNK - NBUF, NCHUNK):
        wait_store(g, g % NBUF)


def kernel(x, token_table, pos_table):
    x_flat = x.reshape(TOTAL).astype(jnp.int32)
    pos2 = pos_table.reshape(MAXLEN // 2, 2 * EMBED)
    out = _embed_sc(x_flat, token_table, pos2)
    return out.reshape(BATCH, MAXLEN, EMBED)


# speedup vs baseline: 1.0942x; 1.0044x over previous
import functools

import jax
import jax.numpy as jnp
from jax import lax
from jax.experimental import pallas as pl
from jax.experimental.pallas import tpu as pltpu
from jax.experimental.pallas import tpu_sc as plsc

VOCAB = 1000000
MAXLEN = 200
EMBED = 64
BATCH = 4096

NUM_CORES = 2
NUM_SUBCORES = 16
NW = NUM_CORES * NUM_SUBCORES
TOTAL = BATCH * MAXLEN
PER_W = TOTAL // NW
CHUNK = 128
NCHUNK = PER_W // CHUNK
NBUF = 5
PREFETCH = 2
LANES = 16
VPR = EMBED // LANES
NGROUP = CHUNK // LANES


_mesh = plsc.VectorSubcoreMesh(core_axis_name="c", subcore_axis_name="s")


@functools.partial(
    pl.kernel,
    out_type=jax.ShapeDtypeStruct((TOTAL, EMBED), jnp.float32),
    mesh=_mesh,
    scratch_types=[
        pltpu.VMEM((PER_W,), jnp.int32),
        pltpu.VMEM((MAXLEN // 2, 2 * EMBED), jnp.float32),
        pltpu.VMEM((NBUF, CHUNK, EMBED), jnp.float32),
        pltpu.SemaphoreType.DMA((NBUF,)),
        pltpu.SemaphoreType.DMA((NBUF,)),
    ],
    compiler_params=pltpu.CompilerParams(use_tc_tiling_on_sc=True),
)
def _embed_sc(x_hbm, tok_hbm, pos_hbm, out_hbm, idx_v, pos_v, bufs, gsem, ssem):
    wid = lax.axis_index("s") * NUM_CORES + lax.axis_index("c")
    base = wid * PER_W

    pltpu.sync_copy(pos_hbm, pos_v)
    pltpu.sync_copy(x_hbm.at[pl.ds(base, PER_W)], idx_v)

    def fetch_group(g, b, u, row0, lo):
        tv = idx_v[pl.ds(g * CHUNK + row0, LANES)]
        for j in range(lo, LANES):
            tok = tv[j]
            pltpu.async_copy(tok_hbm.at[pl.ds(tok, 1), :],
                             bufs.at[b, pl.ds(row0 + j, 1), :],
                             gsem.at[b])

    def start_fetch(g, b):
        @pl.loop(0, NGROUP)
        def _(u):
            fetch_group(g, b, u, u * LANES, 0)

    def wait_fetch(b):
        pltpu.make_async_copy(tok_hbm.at[pl.ds(0, CHUNK), :],
                              bufs.at[b], gsem.at[b]).wait()

    def start_store(g, b):
        pltpu.async_copy(
            bufs.at[b], out_hbm.at[pl.ds(base + g * CHUNK, CHUNK), :], ssem.at[b])

    def wait_store(g, b):
        pltpu.make_async_copy(
            bufs.at[b], out_hbm.at[pl.ds(base + g * CHUNK, CHUNK), :],
            ssem.at[b]).wait()

    def add_pos(g, b):
        p0h = lax.rem(g * (CHUNK // 2), MAXLEN // 2)
        k1h = jnp.minimum((MAXLEN // 2) - p0h, CHUNK // 2)

        def body(rr, prow):
            for half in range(2):
                for v in range(VPR):
                    plsc.addupdate(
                        bufs.at[b, 2 * rr + half, pl.ds(v * LANES, LANES)],
                        pos_v[prow, pl.ds(half * EMBED + v * LANES, LANES)])

        @pl.loop(0, k1h)
        def _(rr):
            body(rr, p0h + rr)

        @pl.loop(k1h, CHUNK // 2)
        def _(rr):
            body(rr, p0h + rr - MAXLEN // 2)

    for b in range(PREFETCH):
        start_fetch(b, b)

    NMAIN = (NCHUNK // NBUF) * NBUF

    @pl.loop(0, NCHUNK // NBUF)
    def _(i):
        for b in range(NBUF):
            g = i * NBUF + b
            h = g + PREFETCH
            bh = (b + PREFETCH) % NBUF

            @pl.when(h < NCHUNK)
            def _():
                @pl.when(h >= NBUF)
                def _():
                    wait_store(h - NBUF, bh)

                start_fetch(h, bh)

            wait_fetch(b)
            add_pos(g, b)
            start_store(g, b)

    for g in range(NMAIN, NCHUNK):
        b = g % NBUF
        h = g + PREFETCH
        if h < NCHUNK:
            wait_store(h - NBUF, h % NBUF)
            start_fetch(h, h % NBUF)
        wait_fetch(b)
        add_pos(g, b)
        start_store(g, b)

    for g in range(NCHUNK - NBUF, NCHUNK):
        wait_store(g, g % NBUF)


def kernel(x, token_table, pos_table):
    x_flat = x.reshape(TOTAL).astype(jnp.int32)
    pos2 = pos_table.reshape(MAXLEN // 2, 2 * EMBED)
    out = _embed_sc(x_flat, token_table, pos2)
    return out.reshape(BATCH, MAXLEN, EMBED)
